# baseline (speedup 1.0000x reference)
import functools

import jax
import jax.numpy as jnp
from jax import lax
from jax.experimental import pallas as pl
from jax.experimental.pallas import tpu as pltpu
from jax.experimental.pallas import tpu_sc as plsc

_N = 10000
_NP = 10240
_E = 160000
_RPT = _NP // 16
_BM = 400
_GM = _N // _BM


def _mesh():
    return plsc.VectorSubcoreMesh(core_axis_name="c", subcore_axis_name="s")


def _zero_fill_2d(ref, nrows, ncols):
    def body(i, _):
        for j in range(ncols // 16):
            ref[i, pl.ds(j * 16, 16)] = jnp.zeros((16,), jnp.float32)
        return 0
    lax.fori_loop(0, nrows, body, 0)


def _make_degrees(B):
    WPT = (_E // B) // 16

    @functools.partial(
        pl.kernel, mesh=_mesh(),
        out_type=jax.ShapeDtypeStruct((2 * _NP,), jnp.float32),
        scratch_types=[
            pltpu.VMEM((WPT, B), jnp.int32),
            pltpu.VMEM((B,), jnp.float32),
            pltpu.VMEM((_RPT,), jnp.float32),
            pltpu.VMEM_SHARED((_NP,), jnp.float32),
            pltpu.SemaphoreType.DMA,
        ],
    )
    def deg_k(ei_hbm, out_hbm, idxv, ones, zbuf, acc, sem):
        cid = lax.axis_index("c")
        sid = lax.axis_index("s")
        for j in range(B // 16):
            ones[pl.ds(j * 16, 16)] = jnp.ones((16,), jnp.float32)
        for j in range(_RPT // 16):
            zbuf[pl.ds(j * 16, 16)] = jnp.zeros((16,), jnp.float32)
        pltpu.sync_copy(zbuf, acc.at[pl.ds(sid * _RPT, _RPT)])
        pltpu.sync_copy(ei_hbm.at[cid * 16 + sid], idxv)
        plsc.subcore_barrier()

        def w_body(w, _):
            pltpu.sync_copy(ones, acc.at[idxv.at[w]], add=True)
            return 0
        lax.fori_loop(0, WPT, w_body, 0)
        plsc.subcore_barrier()

        @pl.when(sid == 0)
        def _():
            pltpu.sync_copy(acc, out_hbm.at[pl.ds(cid * _NP, _NP)])

    return deg_k


def _agg_pass(h_hbm, srcv, dstv, rows, gs, ss, acc, WPT):
    NB = len(rows)
    for k in range(NB):
        pltpu.async_copy(h_hbm.at[srcv.at[k]], rows[k], gs[k])
    NG = WPT // NB

    def body(i, _):
        w0 = NB * i
        for k in range(NB):
            w = w0 + k
            pltpu.make_async_copy(h_hbm.at[srcv.at[w]], rows[k], gs[k]).wait()
            pltpu.async_copy(rows[k], acc.at[dstv.at[w]], ss[k], add=True)

            @pl.when(w + NB < WPT)
            def _(k=k, w=w):
                pltpu.make_async_copy(h_hbm.at[srcv.at[w]], rows[k], ss[k]).wait()
                pltpu.async_copy(h_hbm.at[srcv.at[w + NB]], rows[k], gs[k])
        return 0
    lax.fori_loop(0, NG, body, 0)
    for w in range(NG * NB, WPT):
        k = w % NB
        pltpu.make_async_copy(h_hbm.at[srcv.at[w]], rows[k], gs[k]).wait()
        pltpu.async_copy(rows[k], acc.at[dstv.at[w]], ss[k], add=True)
    for k in range(NB):
        pltpu.make_async_copy(h_hbm.at[srcv.at[0]], rows[k], ss[k]).wait()


def _zero_own_rows(zsrc, acc, sid):
    for r in range(_RPT // 80):
        pltpu.sync_copy(zsrc.at[pl.ds(0, 80)],
                        acc.at[pl.ds(sid * _RPT + r * 80, 80)])


_NBUF = 3
_AGG_SCRATCH = lambda SUBW, B, C: [
    pltpu.VMEM((SUBW, B), jnp.int32),
    pltpu.VMEM((SUBW, B), jnp.int32),
] + [pltpu.VMEM((B, C), jnp.float32) for _ in range(_NBUF)] + [
    pltpu.VMEM_SHARED((_NP, C), jnp.float32),
] + [pltpu.SemaphoreType.DMA for _ in range(2 * _NBUF)]


def _make_agg(NCH, C, B, NSUB, SUBW):

    @functools.partial(
        pl.kernel, mesh=_mesh(),
        out_type=jax.ShapeDtypeStruct((NCH * _NP, C), jnp.float32),
        scratch_types=_AGG_SCRATCH(SUBW, B, C),
    )
    def agg_k(h_hbm, src_hbm, dst_hbm, out_hbm, srcv, dstv, *rest):
        rows, rest = list(rest[:_NBUF]), rest[_NBUF:]
        acc = rest[0]
        gs = list(rest[1:1 + _NBUF])
        ss = list(rest[1 + _NBUF:1 + 2 * _NBUF])
        cid = lax.axis_index("c")
        sid = lax.axis_index("s")
        for cc in range(NCH // 2):
            ch = cid + 2 * cc
            _zero_fill_2d(rows[0], B, C)
            _zero_own_rows(rows[0], acc, sid)
            plsc.subcore_barrier()
            hch = h_hbm.at[pl.ds(ch * _N, _N)]
            for q in range(NSUB):
                pltpu.sync_copy(src_hbm.at[sid, q], srcv)
                pltpu.sync_copy(dst_hbm.at[sid, q], dstv)
                _agg_pass(hch, srcv, dstv, rows, gs, ss, acc, SUBW)
            plsc.subcore_barrier()
            pltpu.sync_copy(acc.at[pl.ds(sid * _RPT, _RPT)],
                            out_hbm.at[pl.ds(ch * _NP + sid * _RPT, _RPT)])

    return agg_k


def _make_agg_esplit(C, B, NSUB, SUBW):

    @functools.partial(
        pl.kernel, mesh=_mesh(),
        out_type=jax.ShapeDtypeStruct((2 * _NP, C), jnp.float32),
        scratch_types=_AGG_SCRATCH(SUBW, B, C),
    )
    def agg_k(h_hbm, src_hbm, dst_hbm, out_hbm, srcv, dstv, *rest):
        rows, rest = list(rest[:_NBUF]), rest[_NBUF:]
        acc = rest[0]
        gs = list(rest[1:1 + _NBUF])
        ss = list(rest[1 + _NBUF:1 + 2 * _NBUF])
        cid = lax.axis_index("c")
        sid = lax.axis_index("s")
        g = cid * 16 + sid
        _zero_fill_2d(rows[0], B, C)
        _zero_own_rows(rows[0], acc, sid)
        plsc.subcore_barrier()
        for q in range(NSUB):
            pltpu.sync_copy(src_hbm.at[g, q], srcv)
            pltpu.sync_copy(dst_hbm.at[g, q], dstv)
            _agg_pass(h_hbm, srcv, dstv, rows, gs, ss, acc, SUBW)
        plsc.subcore_barrier()
        pltpu.sync_copy(acc.at[pl.ds(sid * _RPT, _RPT)],
                        out_hbm.at[pl.ds(cid * _NP + sid * _RPT, _RPT)])

    return agg_k


_deg_call = _make_degrees(80)
_agg2_call = _make_agg(2, 128, 80, 5, 25)
_agg4_call = _make_agg(4, 128, 80, 5, 25)
_aggz_call = _make_agg_esplit(128, 100, 2, 25)


def _tc0(degT, x):
    BM = 1000

    def body(deg_ref, x_ref, h_ref, ns_ref, nd_ref):
        ns = lax.rsqrt(jnp.maximum(deg_ref[:, 0:1], 1.0))
        nd = lax.rsqrt(jnp.maximum(deg_ref[:, 1:2], 1.0))
        h_ref[0] = x_ref[:, :128] * ns
        h_ref[1] = x_ref[:, 128:] * ns
        ns_ref[...] = ns
        nd_ref[...] = nd

    return pl.pallas_call(
        body,
        grid=(_N // BM,),
        in_specs=[
            pl.BlockSpec((BM, 2), lambda m: (m, 0)),
            pl.BlockSpec((BM, 256), lambda m: (m, 0)),
        ],
        out_specs=[
            pl.BlockSpec((2, BM, 128), lambda m: (0, m, 0)),
            pl.BlockSpec((BM, 1), lambda m: (m, 0)),
            pl.BlockSpec((BM, 1), lambda m: (m, 0)),
        ],
        out_shape=[
            jax.ShapeDtypeStruct((2, _N, 128), jnp.float32),
            jax.ShapeDtypeStruct((_N, 1), jnp.float32),
            jax.ShapeDtypeStruct((_N, 1), jnp.float32),
        ],
    )(degT, x)


def _tc1(agg0, W0r, b0, nd, ns):
    BM = 1000

    def body(a_ref, w_ref, b_ref, nd_ref, ns_ref, o_ref):
        t = jnp.dot(a_ref[0], w_ref[0], preferred_element_type=jnp.float32)
        t = t + jnp.dot(a_ref[1], w_ref[1], preferred_element_type=jnp.float32)
        y = jnp.maximum(t * nd_ref[...] + b_ref[...], 0.0) * ns_ref[...]
        for c in range(4):
            o_ref[c] = y[:, 128 * c:128 * (c + 1)]

    return pl.pallas_call(
        body,
        grid=(_N // BM,),
        in_specs=[
            pl.BlockSpec((2, BM, 128), lambda m: (0, m, 0)),
            pl.BlockSpec((2, 128, 512), lambda m: (0, 0, 0)),
            pl.BlockSpec((1, 512), lambda m: (0, 0)),
            pl.BlockSpec((BM, 1), lambda m: (m, 0)),
            pl.BlockSpec((BM, 1), lambda m: (m, 0)),
        ],
        out_specs=pl.BlockSpec((4, BM, 128), lambda m: (0, m, 0)),
        out_shape=jax.ShapeDtypeStruct((4, _N, 128), jnp.float32),
    )(agg0, W0r, b0, nd, ns)


def _tc2(agg1, W1r, b1, W2p, nd, ns):
    BM = 1000

    def body(a_ref, w1_ref, b1_ref, w2_ref, nd_ref, ns_ref, o_ref):
        t = jnp.dot(a_ref[0], w1_ref[0], preferred_element_type=jnp.float32)
        for c in range(1, 4):
            t = t + jnp.dot(a_ref[c], w1_ref[c], preferred_element_type=jnp.float32)
        h = jnp.maximum(t * nd_ref[...] + b1_ref[...], 0.0) * ns_ref[...]
        o_ref[...] = jnp.dot(h, w2_ref[...], preferred_element_type=jnp.float32)

    return pl.pallas_call(
        body,
        grid=(_N // BM,),
        in_specs=[
            pl.BlockSpec((4, BM, 128), lambda m: (0, m, 0)),
            pl.BlockSpec((4, 128, 512), lambda m: (0, 0, 0)),
            pl.BlockSpec((1, 512), lambda m: (0, 0)),
            pl.BlockSpec((512, 128), lambda m: (0, 0)),
            pl.BlockSpec((BM, 1), lambda m: (m, 0)),
            pl.BlockSpec((BM, 1), lambda m: (m, 0)),
        ],
        out_specs=pl.BlockSpec((BM, 128), lambda m: (m, 0)),
        out_shape=jax.ShapeDtypeStruct((_N, 128), jnp.float32),
    )(agg1, W1r, b1, W2p, nd, ns)


def _tc3(parts, nd, b2):
    BM = 2000

    def body(p_ref, nd_ref, b_ref, o_ref):
        t = p_ref[0] + p_ref[1]
        o_ref[...] = t[:, :64] * nd_ref[...] + b_ref[...]

    return pl.pallas_call(
        body,
        grid=(_N // BM,),
        in_specs=[
            pl.BlockSpec((2, BM, 128), lambda m: (0, m, 0)),
            pl.BlockSpec((BM, 1), lambda m: (m, 0)),
            pl.BlockSpec((1, 64), lambda m: (0, 0)),
        ],
        out_specs=pl.BlockSpec((BM, 64), lambda m: (m, 0)),
        out_shape=jax.ShapeDtypeStruct((_N, 64), jnp.float32),
    )(parts, nd, b2)


def kernel(inputs, edge_index, W0, b0, W1, b1, W2, b2):
    ei = edge_index.astype(jnp.int32)
    src2d = ei[0].reshape(16, 5, 25, 80)
    dst3d = ei[1].reshape(16, 5, 25, 80)

    deg = _deg_call(ei.reshape(32, _E // (16 * 80), 80))
    degT = deg.reshape(2, _NP)[:, :_N].T

    h0s3, ns, nd = _tc0(degT, inputs)

    agg0 = _agg2_call(h0s3.reshape(2 * _N, 128), src2d, dst3d)

    h1s3 = _tc1(agg0.reshape(2, _NP, 128), W0.reshape(2, 128, 512),
                b0.reshape(1, 512), nd, ns)

    agg1 = _agg4_call(h1s3.reshape(4 * _N, 128), src2d, dst3d)

    z = _tc2(agg1.reshape(4, _NP, 128), W1.reshape(4, 128, 512),
             b1.reshape(1, 512), jnp.pad(W2, ((0, 0), (0, 64))), nd, ns)

    parts = _aggz_call(z, ei[0].reshape(32, 2, 25, 100),
                       ei[1].reshape(32, 2, 25, 100))
    return _tc3(parts.reshape(2, _NP, 128), nd, b2.reshape(1, 64))


# speedup vs baseline: 1.0278x; 1.0278x over previous
import functools

import jax
import jax.numpy as jnp
from jax import lax
from jax.experimental import pallas as pl
from jax.experimental.pallas import tpu as pltpu
from jax.experimental.pallas import tpu_sc as plsc

_N = 10000
_NP = 10240
_E = 160000
_RPT = _NP // 16
_BM = 400
_GM = _N // _BM


def _mesh():
    return plsc.VectorSubcoreMesh(core_axis_name="c", subcore_axis_name="s")


def _zero_fill_2d(ref, nrows, ncols):
    def body(i, _):
        for j in range(ncols // 16):
            ref[i, pl.ds(j * 16, 16)] = jnp.zeros((16,), jnp.float32)
        return 0
    lax.fori_loop(0, nrows, body, 0)


def _make_degrees(B):
    WPT = (_E // B) // 16

    @functools.partial(
        pl.kernel, mesh=_mesh(),
        out_type=jax.ShapeDtypeStruct((2 * _NP,), jnp.float32),
        scratch_types=[
            pltpu.VMEM((WPT, B), jnp.int32),
            pltpu.VMEM((B,), jnp.float32),
            pltpu.VMEM((_RPT,), jnp.float32),
            pltpu.VMEM_SHARED((_NP,), jnp.float32),
            pltpu.SemaphoreType.DMA,
        ],
    )
    def deg_k(ei_hbm, out_hbm, idxv, ones, zbuf, acc, sem):
        cid = lax.axis_index("c")
        sid = lax.axis_index("s")
        for j in range(B // 16):
            ones[pl.ds(j * 16, 16)] = jnp.ones((16,), jnp.float32)
        for j in range(_RPT // 16):
            zbuf[pl.ds(j * 16, 16)] = jnp.zeros((16,), jnp.float32)
        pltpu.sync_copy(zbuf, acc.at[pl.ds(sid * _RPT, _RPT)])
        pltpu.sync_copy(ei_hbm.at[cid * 16 + sid], idxv)
        plsc.subcore_barrier()

        def w_body(w, _):
            pltpu.sync_copy(ones, acc.at[idxv.at[w]], add=True)
            return 0
        lax.fori_loop(0, WPT, w_body, 0)
        plsc.subcore_barrier()

        @pl.when(sid == 0)
        def _():
            pltpu.sync_copy(acc, out_hbm.at[pl.ds(cid * _NP, _NP)])

    return deg_k


def _agg_pass(h_hbm, srcv, dstv, rows, gs, ss, acc, WPT):
    NB = len(rows)
    for k in range(NB):
        pltpu.async_copy(h_hbm.at[srcv.at[k]], rows[k], gs[k])
    NG = WPT // NB

    def body(i, _):
        w0 = NB * i
        for k in range(NB):
            w = w0 + k
            pltpu.make_async_copy(h_hbm.at[srcv.at[w]], rows[k], gs[k]).wait()
            pltpu.async_copy(rows[k], acc.at[dstv.at[w]], ss[k], add=True)

            @pl.when(w + NB < WPT)
            def _(k=k, w=w):
                pltpu.make_async_copy(h_hbm.at[srcv.at[w]], rows[k], ss[k]).wait()
                pltpu.async_copy(h_hbm.at[srcv.at[w + NB]], rows[k], gs[k])
        return 0
    lax.fori_loop(0, NG, body, 0)
    for w in range(NG * NB, WPT):
        k = w % NB
        pltpu.make_async_copy(h_hbm.at[srcv.at[w]], rows[k], gs[k]).wait()
        pltpu.async_copy(rows[k], acc.at[dstv.at[w]], ss[k], add=True)
    for k in range(NB):
        pltpu.make_async_copy(h_hbm.at[srcv.at[0]], rows[k], ss[k]).wait()


def _zero_own_rows(zsrc, acc, sid):
    for r in range(_RPT // 40):
        pltpu.sync_copy(zsrc.at[pl.ds(0, 40)],
                        acc.at[pl.ds(sid * _RPT + r * 40, 40)])


_NBUF = 4
_AGG_SCRATCH = lambda SUBW, B, C: [
    pltpu.VMEM((SUBW, B), jnp.int32),
    pltpu.VMEM((SUBW, B), jnp.int32),
] + [pltpu.VMEM((B, C), jnp.float32) for _ in range(_NBUF)] + [
    pltpu.VMEM_SHARED((_NP, C), jnp.float32),
] + [pltpu.SemaphoreType.DMA for _ in range(2 * _NBUF)]


def _make_agg(NCH, C, B, NSUB, SUBW):

    @functools.partial(
        pl.kernel, mesh=_mesh(),
        out_type=jax.ShapeDtypeStruct((NCH * _NP, C), jnp.float32),
        scratch_types=_AGG_SCRATCH(SUBW, B, C),
    )
    def agg_k(h_hbm, src_hbm, dst_hbm, out_hbm, srcv, dstv, *rest):
        rows, rest = list(rest[:_NBUF]), rest[_NBUF:]
        acc = rest[0]
        gs = list(rest[1:1 + _NBUF])
        ss = list(rest[1 + _NBUF:1 + 2 * _NBUF])
        cid = lax.axis_index("c")
        sid = lax.axis_index("s")
        for cc in range(NCH // 2):
            ch = cid + 2 * cc
            _zero_fill_2d(rows[0], B, C)
            _zero_own_rows(rows[0], acc, sid)
            plsc.subcore_barrier()
            hch = h_hbm.at[pl.ds(ch * _N, _N)]
            for q in range(NSUB):
                pltpu.sync_copy(src_hbm.at[sid, q], srcv)
                pltpu.sync_copy(dst_hbm.at[sid, q], dstv)
                _agg_pass(hch, srcv, dstv, rows, gs, ss, acc, SUBW)
            plsc.subcore_barrier()
            pltpu.sync_copy(acc.at[pl.ds(sid * _RPT, _RPT)],
                            out_hbm.at[pl.ds(ch * _NP + sid * _RPT, _RPT)])

    return agg_k


def _make_agg_esplit(C, B, NSUB, SUBW):

    @functools.partial(
        pl.kernel, mesh=_mesh(),
        out_type=jax.ShapeDtypeStruct((2 * _NP, C), jnp.float32),
        scratch_types=_AGG_SCRATCH(SUBW, B, C),
    )
    def agg_k(h_hbm, src_hbm, dst_hbm, out_hbm, srcv, dstv, *rest):
        rows, rest = list(rest[:_NBUF]), rest[_NBUF:]
        acc = rest[0]
        gs = list(rest[1:1 + _NBUF])
        ss = list(rest[1 + _NBUF:1 + 2 * _NBUF])
        cid = lax.axis_index("c")
        sid = lax.axis_index("s")
        g = cid * 16 + sid
        _zero_fill_2d(rows[0], B, C)
        _zero_own_rows(rows[0], acc, sid)
        plsc.subcore_barrier()
        for q in range(NSUB):
            pltpu.sync_copy(src_hbm.at[g, q], srcv)
            pltpu.sync_copy(dst_hbm.at[g, q], dstv)
            _agg_pass(h_hbm, srcv, dstv, rows, gs, ss, acc, SUBW)
        plsc.subcore_barrier()
        pltpu.sync_copy(acc.at[pl.ds(sid * _RPT, _RPT)],
                        out_hbm.at[pl.ds(cid * _NP + sid * _RPT, _RPT)])

    return agg_k


_deg_call = _make_degrees(80)
_agg2_call = _make_agg(2, 128, 50, 4, 50)
_agg4_call = _make_agg(4, 128, 50, 4, 50)
_aggz_call = _make_agg_esplit(128, 50, 2, 50)


def _tc0(degT, x):
    BM = 1000

    def body(deg_ref, x_ref, h_ref, ns_ref, nd_ref):
        ns = lax.rsqrt(jnp.maximum(deg_ref[:, 0:1], 1.0))
        nd = lax.rsqrt(jnp.maximum(deg_ref[:, 1:2], 1.0))
        h_ref[0] = x_ref[:, :128] * ns
        h_ref[1] = x_ref[:, 128:] * ns
        ns_ref[...] = ns
        nd_ref[...] = nd

    return pl.pallas_call(
        body,
        grid=(_N // BM,),
        in_specs=[
            pl.BlockSpec((BM, 2), lambda m: (m, 0)),
            pl.BlockSpec((BM, 256), lambda m: (m, 0)),
        ],
        out_specs=[
            pl.BlockSpec((2, BM, 128), lambda m: (0, m, 0)),
            pl.BlockSpec((BM, 1), lambda m: (m, 0)),
            pl.BlockSpec((BM, 1), lambda m: (m, 0)),
        ],
        out_shape=[
            jax.ShapeDtypeStruct((2, _N, 128), jnp.float32),
            jax.ShapeDtypeStruct((_N, 1), jnp.float32),
            jax.ShapeDtypeStruct((_N, 1), jnp.float32),
        ],
    )(degT, x)


def _tc1(agg0, W0r, b0, nd, ns):
    BM = 1000

    def body(a_ref, w_ref, b_ref, nd_ref, ns_ref, o_ref):
        t = jnp.dot(a_ref[0], w_ref[0], preferred_element_type=jnp.float32)
        t = t + jnp.dot(a_ref[1], w_ref[1], preferred_element_type=jnp.float32)
        y = jnp.maximum(t * nd_ref[...] + b_ref[...], 0.0) * ns_ref[...]
        for c in range(4):
            o_ref[c] = y[:, 128 * c:128 * (c + 1)]

    return pl.pallas_call(
        body,
        grid=(_N // BM,),
        in_specs=[
            pl.BlockSpec((2, BM, 128), lambda m: (0, m, 0)),
            pl.BlockSpec((2, 128, 512), lambda m: (0, 0, 0)),
            pl.BlockSpec((1, 512), lambda m: (0, 0)),
            pl.BlockSpec((BM, 1), lambda m: (m, 0)),
            pl.BlockSpec((BM, 1), lambda m: (m, 0)),
        ],
        out_specs=pl.BlockSpec((4, BM, 128), lambda m: (0, m, 0)),
        out_shape=jax.ShapeDtypeStruct((4, _N, 128), jnp.float32),
    )(agg0, W0r, b0, nd, ns)


def _tc2(agg1, W1r, b1, W2p, nd, ns):
    BM = 1000

    def body(a_ref, w1_ref, b1_ref, w2_ref, nd_ref, ns_ref, o_ref):
        t = jnp.dot(a_ref[0], w1_ref[0], preferred_element_type=jnp.float32)
        for c in range(1, 4):
            t = t + jnp.dot(a_ref[c], w1_ref[c], preferred_element_type=jnp.float32)
        h = jnp.maximum(t * nd_ref[...] + b1_ref[...], 0.0) * ns_ref[...]
        o_ref[...] = jnp.dot(h, w2_ref[...], preferred_element_type=jnp.float32)

    return pl.pallas_call(
        body,
        grid=(_N // BM,),
        in_specs=[
            pl.BlockSpec((4, BM, 128), lambda m: (0, m, 0)),
            pl.BlockSpec((4, 128, 512), lambda m: (0, 0, 0)),
            pl.BlockSpec((1, 512), lambda m: (0, 0)),
            pl.BlockSpec((512, 128), lambda m: (0, 0)),
            pl.BlockSpec((BM, 1), lambda m: (m, 0)),
            pl.BlockSpec((BM, 1), lambda m: (m, 0)),
        ],
        out_specs=pl.BlockSpec((BM, 128), lambda m: (m, 0)),
        out_shape=jax.ShapeDtypeStruct((_N, 128), jnp.float32),
    )(agg1, W1r, b1, W2p, nd, ns)


def _tc3(parts, nd, b2):
    BM = 2000

    def body(p_ref, nd_ref, b_ref, o_ref):
        t = p_ref[0] + p_ref[1]
        o_ref[...] = t[:, :64] * nd_ref[...] + b_ref[...]

    return pl.pallas_call(
        body,
        grid=(_N // BM,),
        in_specs=[
            pl.BlockSpec((2, BM, 128), lambda m: (0, m, 0)),
            pl.BlockSpec((BM, 1), lambda m: (m, 0)),
            pl.BlockSpec((1, 64), lambda m: (0, 0)),
        ],
        out_specs=pl.BlockSpec((BM, 64), lambda m: (m, 0)),
        out_shape=jax.ShapeDtypeStruct((_N, 64), jnp.float32),
    )(parts, nd, b2)


def kernel(inputs, edge_index, W0, b0, W1, b1, W2, b2):
    ei = edge_index.astype(jnp.int32)
    src2d = ei[0].reshape(16, 4, 50, 50)
    dst3d = ei[1].reshape(16, 4, 50, 50)

    deg = _deg_call(ei.reshape(32, _E // (16 * 80), 80))
    degT = deg.reshape(2, _NP)[:, :_N].T

    h0s3, ns, nd = _tc0(degT, inputs)

    agg0 = _agg2_call(h0s3.reshape(2 * _N, 128), src2d, dst3d)

    h1s3 = _tc1(agg0.reshape(2, _NP, 128), W0.reshape(2, 128, 512),
                b0.reshape(1, 512), nd, ns)

    agg1 = _agg4_call(h1s3.reshape(4 * _N, 128), src2d, dst3d)

    z = _tc2(agg1.reshape(4, _NP, 128), W1.reshape(4, 128, 512),
             b1.reshape(1, 512), jnp.pad(W2, ((0, 0), (0, 64))), nd, ns)

    parts = _aggz_call(z, ei[0].reshape(32, 2, 50, 50),
                       ei[1].reshape(32, 2, 50, 50))
    return _tc3(parts.reshape(2, _NP, 128), nd, b2.reshape(1, 64))


# speedup vs baseline: 1.0358x; 1.0078x over previous
import functools

import jax
import jax.numpy as jnp
from jax import lax
from jax.experimental import pallas as pl
from jax.experimental.pallas import tpu as pltpu
from jax.experimental.pallas import tpu_sc as plsc

_N = 10000
_NP = 10240
_E = 160000
_RPT = _NP // 16
_BM = 400
_GM = _N // _BM


def _mesh():
    return plsc.VectorSubcoreMesh(core_axis_name="c", subcore_axis_name="s")


def _zero_fill_2d(ref, nrows, ncols):
    def body(i, _):
        for j in range(ncols // 16):
            ref[i, pl.ds(j * 16, 16)] = jnp.zeros((16,), jnp.float32)
        return 0
    lax.fori_loop(0, nrows, body, 0)


def _make_degrees(B):
    WPT = (_E // B) // 16

    @functools.partial(
        pl.kernel, mesh=_mesh(),
        out_type=jax.ShapeDtypeStruct((2 * _NP,), jnp.float32),
        scratch_types=[
            pltpu.VMEM((WPT, B), jnp.int32),
            pltpu.VMEM((B,), jnp.float32),
            pltpu.VMEM((_RPT,), jnp.float32),
            pltpu.VMEM_SHARED((_NP,), jnp.float32),
            pltpu.SemaphoreType.DMA,
        ],
    )
    def deg_k(ei_hbm, out_hbm, idxv, ones, zbuf, acc, sem):
        cid = lax.axis_index("c")
        sid = lax.axis_index("s")
        for j in range(B // 16):
            ones[pl.ds(j * 16, 16)] = jnp.ones((16,), jnp.float32)
        for j in range(_RPT // 16):
            zbuf[pl.ds(j * 16, 16)] = jnp.zeros((16,), jnp.float32)
        pltpu.sync_copy(zbuf, acc.at[pl.ds(sid * _RPT, _RPT)])
        pltpu.sync_copy(ei_hbm.at[cid * 16 + sid], idxv)
        plsc.subcore_barrier()

        def w_body(w, _):
            pltpu.async_copy(ones, acc.at[idxv.at[w]], sem, add=True)
            return 0
        lax.fori_loop(0, WPT, w_body, 0)

        def w_drain(w, _):
            pltpu.make_async_copy(ones, acc.at[idxv.at[w]], sem).wait()
            return 0
        lax.fori_loop(0, WPT, w_drain, 0)
        plsc.subcore_barrier()

        @pl.when(sid == 0)
        def _():
            pltpu.sync_copy(acc, out_hbm.at[pl.ds(cid * _NP, _NP)])

    return deg_k


def _agg_pass(h_hbm, srcv, dstv, rows, gs, ss, acc, WPT):
    NB = len(rows)
    for k in range(NB):
        pltpu.async_copy(h_hbm.at[srcv.at[k]], rows[k], gs[k])
    NG = WPT // NB

    def body(i, _):
        w0 = NB * i
        for k in range(NB):
            w = w0 + k
            pltpu.make_async_copy(h_hbm.at[srcv.at[w]], rows[k], gs[k]).wait()
            pltpu.async_copy(rows[k], acc.at[dstv.at[w]], ss[k], add=True)

            @pl.when(w + NB < WPT)
            def _(k=k, w=w):
                pltpu.make_async_copy(h_hbm.at[srcv.at[w]], rows[k], ss[k]).wait()
                pltpu.async_copy(h_hbm.at[srcv.at[w + NB]], rows[k], gs[k])
        return 0
    lax.fori_loop(0, NG, body, 0)
    for w in range(NG * NB, WPT):
        k = w % NB
        pltpu.make_async_copy(h_hbm.at[srcv.at[w]], rows[k], gs[k]).wait()
        pltpu.async_copy(rows[k], acc.at[dstv.at[w]], ss[k], add=True)
    for k in range(NB):
        pltpu.make_async_copy(h_hbm.at[srcv.at[0]], rows[k], ss[k]).wait()


def _zero_own_rows(zsrc, acc, sid):
    for r in range(_RPT // 40):
        pltpu.sync_copy(zsrc.at[pl.ds(0, 40)],
                        acc.at[pl.ds(sid * _RPT + r * 40, 40)])


_NBUF = 4
_AGG_SCRATCH = lambda SUBW, B, C: [
    pltpu.VMEM((SUBW, B), jnp.int32),
    pltpu.VMEM((SUBW, B), jnp.int32),
] + [pltpu.VMEM((B, C), jnp.float32) for _ in range(_NBUF)] + [
    pltpu.VMEM_SHARED((_NP, C), jnp.float32),
] + [pltpu.SemaphoreType.DMA for _ in range(2 * _NBUF)]


def _make_agg(NCH, C, B, NSUB, SUBW):

    @functools.partial(
        pl.kernel, mesh=_mesh(),
        out_type=jax.ShapeDtypeStruct((NCH * _NP, C), jnp.float32),
        scratch_types=_AGG_SCRATCH(SUBW, B, C),
    )
    def agg_k(h_hbm, src_hbm, dst_hbm, out_hbm, srcv, dstv, *rest):
        rows, rest = list(rest[:_NBUF]), rest[_NBUF:]
        acc = rest[0]
        gs = list(rest[1:1 + _NBUF])
        ss = list(rest[1 + _NBUF:1 + 2 * _NBUF])
        cid = lax.axis_index("c")
        sid = lax.axis_index("s")
        for cc in range(NCH // 2):
            ch = cid + 2 * cc
            _zero_fill_2d(rows[0], B, C)
            _zero_own_rows(rows[0], acc, sid)
            plsc.subcore_barrier()
            hch = h_hbm.at[pl.ds(ch * _N, _N)]
            for q in range(NSUB):
                pltpu.sync_copy(src_hbm.at[sid, q], srcv)
                pltpu.sync_copy(dst_hbm.at[sid, q], dstv)
                _agg_pass(hch, srcv, dstv, rows, gs, ss, acc, SUBW)
            plsc.subcore_barrier()
            pltpu.sync_copy(acc.at[pl.ds(sid * _RPT, _RPT)],
                            out_hbm.at[pl.ds(ch * _NP + sid * _RPT, _RPT)])

    return agg_k


def _make_agg_esplit(C, B, NSUB, SUBW):

    @functools.partial(
        pl.kernel, mesh=_mesh(),
        out_type=jax.ShapeDtypeStruct((2 * _NP, C), jnp.float32),
        scratch_types=_AGG_SCRATCH(SUBW, B, C),
    )
    def agg_k(h_hbm, src_hbm, dst_hbm, out_hbm, srcv, dstv, *rest):
        rows, rest = list(rest[:_NBUF]), rest[_NBUF:]
        acc = rest[0]
        gs = list(rest[1:1 + _NBUF])
        ss = list(rest[1 + _NBUF:1 + 2 * _NBUF])
        cid = lax.axis_index("c")
        sid = lax.axis_index("s")
        g = cid * 16 + sid
        _zero_fill_2d(rows[0], B, C)
        _zero_own_rows(rows[0], acc, sid)
        plsc.subcore_barrier()
        for q in range(NSUB):
            pltpu.sync_copy(src_hbm.at[g, q], srcv)
            pltpu.sync_copy(dst_hbm.at[g, q], dstv)
            _agg_pass(h_hbm, srcv, dstv, rows, gs, ss, acc, SUBW)
        plsc.subcore_barrier()
        pltpu.sync_copy(acc.at[pl.ds(sid * _RPT, _RPT)],
                        out_hbm.at[pl.ds(cid * _NP + sid * _RPT, _RPT)])

    return agg_k


_deg_call = _make_degrees(80)
_agg2_call = _make_agg(2, 128, 50, 4, 50)
_agg4_call = _make_agg(4, 128, 50, 4, 50)
_aggz_call = _make_agg_esplit(128, 50, 2, 50)


def _tc0(degT, x):
    BM = 1000

    def body(deg_ref, x_ref, h_ref, ns_ref, nd_ref):
        ns = lax.rsqrt(jnp.maximum(deg_ref[:, 0:1], 1.0))
        nd = lax.rsqrt(jnp.maximum(deg_ref[:, 1:2], 1.0))
        h_ref[0] = x_ref[:, :128] * ns
        h_ref[1] = x_ref[:, 128:] * ns
        ns_ref[...] = ns
        nd_ref[...] = nd

    return pl.pallas_call(
        body,
        grid=(_N // BM,),
        in_specs=[
            pl.BlockSpec((BM, 2), lambda m: (m, 0)),
            pl.BlockSpec((BM, 256), lambda m: (m, 0)),
        ],
        out_specs=[
            pl.BlockSpec((2, BM, 128), lambda m: (0, m, 0)),
            pl.BlockSpec((BM, 1), lambda m: (m, 0)),
            pl.BlockSpec((BM, 1), lambda m: (m, 0)),
        ],
        out_shape=[
            jax.ShapeDtypeStruct((2, _N, 128), jnp.float32),
            jax.ShapeDtypeStruct((_N, 1), jnp.float32),
            jax.ShapeDtypeStruct((_N, 1), jnp.float32),
        ],
    )(degT, x)


def _tc1(agg0, W0r, b0, nd, ns):
    BM = 1000

    def body(a_ref, w_ref, b_ref, nd_ref, ns_ref, o_ref):
        t = jnp.dot(a_ref[0], w_ref[0], preferred_element_type=jnp.float32)
        t = t + jnp.dot(a_ref[1], w_ref[1], preferred_element_type=jnp.float32)
        y = jnp.maximum(t * nd_ref[...] + b_ref[...], 0.0) * ns_ref[...]
        for c in range(4):
            o_ref[c] = y[:, 128 * c:128 * (c + 1)]

    return pl.pallas_call(
        body,
        grid=(_N // BM,),
        in_specs=[
            pl.BlockSpec((2, BM, 128), lambda m: (0, m, 0)),
            pl.BlockSpec((2, 128, 512), lambda m: (0, 0, 0)),
            pl.BlockSpec((1, 512), lambda m: (0, 0)),
            pl.BlockSpec((BM, 1), lambda m: (m, 0)),
            pl.BlockSpec((BM, 1), lambda m: (m, 0)),
        ],
        out_specs=pl.BlockSpec((4, BM, 128), lambda m: (0, m, 0)),
        out_shape=jax.ShapeDtypeStruct((4, _N, 128), jnp.float32),
    )(agg0, W0r, b0, nd, ns)


def _tc2(agg1, W1r, b1, W2p, nd, ns):
    BM = 1000

    def body(a_ref, w1_ref, b1_ref, w2_ref, nd_ref, ns_ref, o_ref):
        t = jnp.dot(a_ref[0], w1_ref[0], preferred_element_type=jnp.float32)
        for c in range(1, 4):
            t = t + jnp.dot(a_ref[c], w1_ref[c], preferred_element_type=jnp.float32)
        h = jnp.maximum(t * nd_ref[...] + b1_ref[...], 0.0) * ns_ref[...]
        o_ref[...] = jnp.dot(h, w2_ref[...], preferred_element_type=jnp.float32)

    return pl.pallas_call(
        body,
        grid=(_N // BM,),
        in_specs=[
            pl.BlockSpec((4, BM, 128), lambda m: (0, m, 0)),
            pl.BlockSpec((4, 128, 512), lambda m: (0, 0, 0)),
            pl.BlockSpec((1, 512), lambda m: (0, 0)),
            pl.BlockSpec((512, 128), lambda m: (0, 0)),
            pl.BlockSpec((BM, 1), lambda m: (m, 0)),
            pl.BlockSpec((BM, 1), lambda m: (m, 0)),
        ],
        out_specs=pl.BlockSpec((BM, 128), lambda m: (m, 0)),
        out_shape=jax.ShapeDtypeStruct((_N, 128), jnp.float32),
    )(agg1, W1r, b1, W2p, nd, ns)


def _tc3(parts, nd, b2):
    BM = 2000

    def body(p_ref, nd_ref, b_ref, o_ref):
        t = p_ref[0] + p_ref[1]
        o_ref[...] = t[:, :64] * nd_ref[...] + b_ref[...]

    return pl.pallas_call(
        body,
        grid=(_N // BM,),
        in_specs=[
            pl.BlockSpec((2, BM, 128), lambda m: (0, m, 0)),
            pl.BlockSpec((BM, 1), lambda m: (m, 0)),
            pl.BlockSpec((1, 64), lambda m: (0, 0)),
        ],
        out_specs=pl.BlockSpec((BM, 64), lambda m: (m, 0)),
        out_shape=jax.ShapeDtypeStruct((_N, 64), jnp.float32),
    )(parts, nd, b2)


def kernel(inputs, edge_index, W0, b0, W1, b1, W2, b2):
    ei = edge_index.astype(jnp.int32)
    src2d = ei[0].reshape(16, 4, 50, 50)
    dst3d = ei[1].reshape(16, 4, 50, 50)

    deg = _deg_call(ei.reshape(32, _E // (16 * 80), 80))
    degT = deg.reshape(2, _NP)[:, :_N].T

    h0s3, ns, nd = _tc0(degT, inputs)

    agg0 = _agg2_call(h0s3.reshape(2 * _N, 128), src2d, dst3d)

    h1s3 = _tc1(agg0.reshape(2, _NP, 128), W0.reshape(2, 128, 512),
                b0.reshape(1, 512), nd, ns)

    agg1 = _agg4_call(h1s3.reshape(4 * _N, 128), src2d, dst3d)

    z = _tc2(agg1.reshape(4, _NP, 128), W1.reshape(4, 128, 512),
             b1.reshape(1, 512), jnp.pad(W2, ((0, 0), (0, 64))), nd, ns)

    parts = _aggz_call(z, ei[0].reshape(32, 2, 50, 50),
                       ei[1].reshape(32, 2, 50, 50))
    return _tc3(parts.reshape(2, _NP, 128), nd, b2.reshape(1, 64))
